# baseline (speedup 1.0000x reference)
import functools

import jax
import jax.numpy as jnp
from jax.experimental import pallas as pl

BLOCK = 2048
K = 32
OUT_DIM = 32


def _mlp_body(x_ref, W1_ref, b1_ref, W2_ref, b2_ref, W3_ref, b3_ref, W4_ref, b4_ref, o_ref):
    x = x_ref[...]
    h = jnp.maximum(jnp.dot(x, W1_ref[...], preferred_element_type=jnp.float32) + b1_ref[...], 0.0)
    h = jnp.maximum(jnp.dot(h, W2_ref[...], preferred_element_type=jnp.float32) + b2_ref[...], 0.0)
    h = jnp.maximum(jnp.dot(h, W3_ref[...], preferred_element_type=jnp.float32) + b3_ref[...], 0.0)
    o_ref[...] = jnp.dot(h, W4_ref[...], preferred_element_type=jnp.float32) + b4_ref[...]


def _mlp(x, W1, b1, W2, b2, W3, b3, W4, b4):
    n = x.shape[0]
    return pl.pallas_call(
        _mlp_body,
        out_shape=jax.ShapeDtypeStruct((n, OUT_DIM), jnp.float32),
    )(x, W1, b1.reshape(1, -1), W2, b2.reshape(1, -1), W3, b3.reshape(1, -1), W4, b4.reshape(1, -1))


def _find_knn(x):
    n = x.shape[0]
    bs = n // BLOCK
    dists = []
    index = []
    for b in range(bs):
        start = b * BLOCK
        end = start + BLOCK
        idx = jnp.concatenate([jnp.arange(start), jnp.arange(end, n)], axis=0)
        q = x[start:end]
        keys = x[idx]
        d2 = jnp.sum(q * q, axis=1, keepdims=True) + jnp.sum(keys * keys, axis=1)[None, :] - 2.0 * (q @ keys.T)
        negd, i = jax.lax.top_k(-d2, K)
        dists.append(-negd)
        index.append(idx[i.reshape(-1)].reshape(-1, K))
    return jnp.concatenate(dists, axis=0), jnp.concatenate(index, axis=0)


def _affinity(x):
    n = x.shape[0]
    d, nn_idx = _find_knn(x)
    sigma = d.mean(axis=1, keepdims=True)
    w = jnp.exp(-d ** 2 / (2.0 * sigma ** 2)).reshape(n * K)
    w = jnp.concatenate([w, w], axis=0)
    nn_flat = nn_idx.reshape(n * K)
    rows = jnp.repeat(jnp.arange(n), K)
    row_idx = jnp.concatenate([nn_flat, rows], axis=0)
    col_idx = jnp.concatenate([rows, nn_flat], axis=0)
    return w / 2.0, row_idx, col_idx, nn_flat


def kernel(x1, x2, W1, b1, W2, b2, W3, b3, W4, b4):
    n = x1.shape[0]
    y = _mlp(x1, W1, b1, W2, b2, W3, b3, W4, b4)
    vals1, r1, c1, _ = _affinity(x1)
    deg1 = jax.ops.segment_sum(vals1, r1, num_segments=n)
    y = y / deg1[:, None]
    gram = y.T @ y + 1e-07 * jnp.eye(y.shape[1], dtype=y.dtype)
    l = jnp.linalg.cholesky(gram)
    operator = jax.scipy.linalg.solve_triangular(l, jnp.eye(l.shape[0], dtype=l.dtype), lower=True).T * (y.shape[0] ** 0.5)
    vals2, r2, c2, nn_flat = _affinity(x2)
    y2 = _mlp(x2, W1, b1, W2, b2, W3, b3, W4, b4) @ operator
    deg2 = jax.ops.segment_sum(vals2, r2, num_segments=n)
    y2 = y2 / deg2[:, None]
    diff = y2[:, None, :] - y2[nn_flat].reshape(-1, K, OUT_DIM)
    dd = jnp.sum(diff * diff, axis=-1).reshape(n * K)
    dd = jnp.concatenate([dd, dd], axis=0)
    colsum_a = jax.ops.segment_sum(vals2, c2, num_segments=n)
    rowsum_d = jax.ops.segment_sum(dd, r2, num_segments=n)
    return jnp.dot(colsum_a, rowsum_d) / n


# speedup vs baseline: 1.3320x; 1.3320x over previous
import functools
import math

import jax
import jax.numpy as jnp
from jax.experimental import pallas as pl
from jax.experimental.pallas import tpu as pltpu

BLOCK = 2048
K = 32
OUT_DIM = 32
N = 4096


def _mlp_body(x_ref, W1_ref, b1_ref, W2_ref, b2_ref, W3_ref, b3_ref, W4_ref,
              b4_ref, o_ref):
    x = x_ref[...]
    h = jnp.maximum(jnp.dot(x, W1_ref[...], preferred_element_type=jnp.float32) + b1_ref[...], 0.0)
    h = jnp.maximum(jnp.dot(h, W2_ref[...], preferred_element_type=jnp.float32) + b2_ref[...], 0.0)
    h = jnp.maximum(jnp.dot(h, W3_ref[...], preferred_element_type=jnp.float32) + b3_ref[...], 0.0)
    o_ref[...] = jnp.dot(h, W4_ref[...], preferred_element_type=jnp.float32) + b4_ref[...]


def _mlp(x, W1, b1, W2, b2, W3, b3, W4, b4):
    n = x.shape[0]
    return pl.pallas_call(
        _mlp_body,
        out_shape=jax.ShapeDtypeStruct((n, OUT_DIM), jnp.float32),
    )(x, W1, b1.reshape(1, -1), W2, b2.reshape(1, -1), W3, b3.reshape(1, -1),
      W4, b4.reshape(1, -1))


def _dist_body(q_ref, k_ref, o_ref):
    q = q_ref[...]
    k = k_ref[...]
    g = jax.lax.dot_general(q, k, (((1,), (1,)), ((), ())),
                            preferred_element_type=jnp.float32)
    qq = jnp.sum(q * q, axis=1, keepdims=True)
    kk = jnp.sum(k * k, axis=1, keepdims=True)
    o_ref[...] = qq + kk.reshape(1, -1) - 2.0 * g


def _dist(q, k):
    nq = q.shape[0]
    nk = k.shape[0]
    rb = 512
    return pl.pallas_call(
        _dist_body,
        grid=(nq // rb,),
        in_specs=[
            pl.BlockSpec((rb, q.shape[1]), lambda i: (i, 0)),
            pl.BlockSpec((nk, k.shape[1]), lambda i: (0, 0)),
        ],
        out_specs=pl.BlockSpec((rb, nk), lambda i: (i, 0)),
        out_shape=jax.ShapeDtypeStruct((nq, nk), jnp.float32),
    )(q, k)


def _solve_body(m1_ref, m2_ref, deg1_ref, deg2_ref, y2_ref):
    n = m1_ref.shape[0]
    dim = OUT_DIM
    y = m1_ref[...] / deg1_ref[...]
    gram = jax.lax.dot_general(y, y, (((0,), (0,)), ((), ())),
                               preferred_element_type=jnp.float32)
    iota_r = jax.lax.broadcasted_iota(jnp.int32, (dim, dim), 0)
    iota_c = jax.lax.broadcasted_iota(jnp.int32, (dim, dim), 1)
    gram = gram + jnp.where((iota_r == iota_c), 1e-7, 0.0)

    def chol_step(j, carry):
        A, L = carry
        ajj = jnp.sum(jnp.where((iota_r == j) & (iota_c == j), A, 0.0))
        d = jax.lax.rsqrt(ajj)
        colj = jnp.sum(jnp.where(iota_c == j, A, 0.0), axis=1, keepdims=True)
        c = jnp.where(iota_r[:, :1] >= j, colj * d, 0.0)
        L = L + c * (iota_c == j).astype(jnp.float32)
        A = A - c * c.reshape(1, dim)
        return A, L

    _, L = jax.lax.fori_loop(0, dim, chol_step,
                             (gram, jnp.zeros((dim, dim), jnp.float32)))

    def inv_step(j, T):
        lrow = jnp.sum(jnp.where(iota_r == j, L, 0.0), axis=0, keepdims=True)
        ljj = jnp.sum(jnp.where(iota_c[:1, :] == j, lrow, 0.0))
        below = jnp.where(iota_c[:1, :] < j, lrow, 0.0)
        s = jnp.sum(below.reshape(dim, 1) * T, axis=0, keepdims=True)
        ej = (iota_c[:1, :] == j).astype(jnp.float32)
        rowv = (ej - s) / ljj
        return T + (iota_r == j).astype(jnp.float32) * rowv

    T = jax.lax.fori_loop(0, dim, inv_step, jnp.zeros((dim, dim), jnp.float32))

    z = jax.lax.dot_general(m2_ref[...], T, (((1,), (1,)), ((), ())),
                            preferred_element_type=jnp.float32)
    y2_ref[...] = z * (float(n) ** 0.5) / deg2_ref[...]


def _solve_stage(m1, m2, deg1, deg2):
    n = m1.shape[0]
    return pl.pallas_call(
        _solve_body,
        out_shape=jax.ShapeDtypeStruct((n, OUT_DIM), jnp.float32),
    )(m1, m2, deg1.reshape(n, 1), deg2.reshape(n, 1))


def _affinity_parts(D, DT):
    negd0, i0 = jax.lax.top_k(-D, K)
    negd1, i1 = jax.lax.top_k(-DT, K)
    d = jnp.concatenate([-negd0, -negd1], axis=0)
    nn = jnp.concatenate([i0 + BLOCK, i1], axis=0)
    sigma = d.mean(axis=1, keepdims=True)
    w_half = 0.5 * jnp.exp(-d ** 2 / (2.0 * sigma ** 2))
    rw = w_half.sum(axis=1)
    return w_half, rw, nn


def kernel(x1, x2, W1, b1, W2, b2, W3, b3, W4, b4):
    n = x1.shape[0]
    xs = jnp.concatenate([x1, x2], axis=0)
    m = _mlp(xs, W1, b1, W2, b2, W3, b3, W4, b4)
    m1, m2 = m[:n], m[n:]

    D1 = _dist(x1[:BLOCK], x1[BLOCK:])
    D1T = _dist(x1[BLOCK:], x1[:BLOCK])
    D2 = _dist(x2[:BLOCK], x2[BLOCK:])
    D2T = _dist(x2[BLOCK:], x2[:BLOCK])

    w1, rw1, nn1 = _affinity_parts(D1, D1T)
    w2, rw2, nn2 = _affinity_parts(D2, D2T)

    deg1 = rw1 + jax.ops.segment_sum(w1.reshape(-1), nn1.reshape(-1),
                                     num_segments=n)
    deg2 = rw2 + jax.ops.segment_sum(w2.reshape(-1), nn2.reshape(-1),
                                     num_segments=n)

    y2 = _solve_stage(m1, m2, deg1, deg2)

    nn_flat = nn2.reshape(-1)
    diff = y2[:, None, :] - y2[nn_flat].reshape(n, K, OUT_DIM)
    dd0 = jnp.sum(diff * diff, axis=-1)
    wgt = deg2[:, None] + deg2[nn_flat].reshape(n, K)
    return jnp.sum(dd0 * wgt) / n
